# baseline (speedup 1.0000x reference)
import functools

import jax
import jax.numpy as jnp
from jax import lax
from jax.experimental import pallas as pl
from jax.experimental.pallas import tpu as pltpu
from jax.experimental.pallas import tpu_sc as plsc

NC = 2
NS = 16
NW = NC * NS
CH = 128
TRASH = 8


def _encode_body(x_ref, w_ref, b_ref, o_ref):
    acc = jnp.dot(x_ref[...], w_ref[...], preferred_element_type=jnp.float32)
    o_ref[...] = jnp.maximum(acc + b_ref[...], 0.0)


def _encode(x, w, b2d, block_rows):
    n, d = x.shape
    h = w.shape[1]
    grid = n // block_rows
    return pl.pallas_call(
        _encode_body,
        grid=(grid,),
        in_specs=[
            pl.BlockSpec((block_rows, d), lambda i: (i, 0)),
            pl.BlockSpec((d, h), lambda i: (0, 0)),
            pl.BlockSpec((1, h), lambda i: (0, 0)),
        ],
        out_specs=pl.BlockSpec((block_rows, h), lambda i: (i, 0)),
        out_shape=jax.ShapeDtypeStruct((n, h), jnp.float32),
    )(x, w, b2d)


def _make_sc_segment_sum(n, hdim, nchunk):
    base = (n // NS) // 8 * 8
    rem = n - NS * base
    step = 16
    passes = tuple((i, min(step, nchunk - i)) for i in range(0, nchunk, step))
    idx_rows = step
    mesh = plsc.VectorSubcoreMesh(core_axis_name="c", subcore_axis_name="s")

    @functools.partial(
        pl.kernel,
        mesh=mesh,
        out_type=jax.ShapeDtypeStruct((NC, n, hdim), jnp.float32),
        scratch_types=[
            pltpu.VMEM((idx_rows, CH), jnp.int32),
            pltpu.VMEM((idx_rows, CH), jnp.int32),
            pltpu.VMEM((CH, hdim), jnp.float32),
            pltpu.VMEM((CH, hdim), jnp.float32),
            pltpu.VMEM_SHARED((n + TRASH, hdim), jnp.float32),
            pltpu.SemaphoreType.DMA,
            pltpu.SemaphoreType.DMA,
            pltpu.SemaphoreType.DMA,
            pltpu.SemaphoreType.DMA,
        ],
    )
    def sc_seg_sum(h_hbm, src_hbm, dst_hbm, zeros_hbm, out_hbm,
                   src_v, dst_v, rows0_v, rows1_v, agg_sh, sem0, sem1,
                   ssem0, ssem1):
        c = lax.axis_index("c")
        s = lax.axis_index("s")
        wid = s * NC + c
        row0 = s * base
        pltpu.sync_copy(zeros_hbm.at[pl.ds(row0, base)],
                        agg_sh.at[pl.ds(row0, base)])

        @pl.when(s == NS - 1)
        def _zero_tail():
            pltpu.sync_copy(zeros_hbm.at[pl.ds(NS * base, rem)],
                            agg_sh.at[pl.ds(NS * base, rem)])
        plsc.subcore_barrier()

        bufs = (rows0_v, rows1_v)
        gsems = (sem0, sem1)
        ssems = (ssem0, ssem1)
        nbuf = 2

        for start, cnt in passes:
            pltpu.sync_copy(src_hbm.at[wid, pl.ds(start, cnt)],
                            src_v.at[pl.ds(0, cnt)])
            pltpu.sync_copy(dst_hbm.at[wid, pl.ds(start, cnt)],
                            dst_v.at[pl.ds(0, cnt)])
            for b in range(min(nbuf, cnt)):
                pltpu.async_copy(h_hbm.at[src_v.at[b]], bufs[b], gsems[b])

            def body(t, carry):
                j0 = t * nbuf
                for b in range(nbuf):
                    j = j0 + b
                    pltpu.make_async_copy(h_hbm.at[src_v.at[j]], bufs[b],
                                          gsems[b]).wait()
                    pltpu.async_copy(bufs[b], agg_sh.at[dst_v.at[j]],
                                     ssems[b], add=True)

                    @pl.when(j + nbuf < cnt)
                    def _next():
                        pltpu.make_async_copy(bufs[b],
                                              agg_sh.at[dst_v.at[j]],
                                              ssems[b]).wait()
                        pltpu.async_copy(h_hbm.at[src_v.at[j + nbuf]],
                                         bufs[b], gsems[b])
                return carry

            lax.fori_loop(0, cnt // nbuf, body, 0)
            for j in range(cnt // nbuf * nbuf, cnt):
                b = j % nbuf
                pltpu.make_async_copy(h_hbm.at[src_v.at[j]], bufs[b],
                                      gsems[b]).wait()
                pltpu.async_copy(bufs[b], agg_sh.at[dst_v.at[j]],
                                 ssems[b], add=True)
            for j in range(max(0, cnt - nbuf), cnt):
                b = j % nbuf
                pltpu.make_async_copy(bufs[b], agg_sh.at[dst_v.at[j]],
                                      ssems[b]).wait()
        plsc.subcore_barrier()
        pltpu.sync_copy(agg_sh.at[pl.ds(row0, base)],
                        out_hbm.at[c, pl.ds(row0, base)])

        @pl.when(s == NS - 1)
        def _out_tail():
            pltpu.sync_copy(agg_sh.at[pl.ds(NS * base, rem)],
                            out_hbm.at[c, pl.ds(NS * base, rem)])

    return sc_seg_sum


def _final_body(p_ref, h_ref, wm_ref, bm_ref, w1_ref, b1_ref, w2_ref, b2_ref,
                o_ref):
    agg = p_ref[0] + p_ref[1]
    t = jnp.dot(agg, wm_ref[...], preferred_element_type=jnp.float32)
    t = jnp.maximum(t + bm_ref[...], 0.0) + h_ref[...]
    hid = jnp.dot(t, w1_ref[...], preferred_element_type=jnp.float32)
    hid = jnp.maximum(hid + b1_ref[...], 0.0)
    o_ref[...] = jnp.dot(hid, w2_ref[...],
                         preferred_element_type=jnp.float32) + b2_ref[...]


def _final(parts, h, wm, bm2d, w1, b12d, w2, b22d, block_rows):
    n, hdim = h.shape
    grid = n // block_rows
    return pl.pallas_call(
        _final_body,
        grid=(grid,),
        in_specs=[
            pl.BlockSpec((NC, block_rows, hdim), lambda i: (0, i, 0)),
            pl.BlockSpec((block_rows, hdim), lambda i: (i, 0)),
            pl.BlockSpec((hdim, hdim), lambda i: (0, 0)),
            pl.BlockSpec((1, hdim), lambda i: (0, 0)),
            pl.BlockSpec((hdim, hdim), lambda i: (0, 0)),
            pl.BlockSpec((1, hdim), lambda i: (0, 0)),
            pl.BlockSpec((hdim, 1), lambda i: (0, 0)),
            pl.BlockSpec((1, 1), lambda i: (0, 0)),
        ],
        out_specs=pl.BlockSpec((block_rows, 1), lambda i: (i, 0)),
        out_shape=jax.ShapeDtypeStruct((n, 1), jnp.float32),
    )(parts, h, wm, bm2d, w1, b12d, w2, b22d)


def kernel(x, edge_index, W_enc, b_enc, W_msg, b_msg, W_out1, b_out1,
           W_out2, b_out2):
    n, d = x.shape
    hdim = W_enc.shape[1]
    e = edge_index.shape[1]
    nchunk = -(-e // (NW * CH))

    h = _encode(x, W_enc, b_enc.reshape(1, hdim), block_rows=1000)

    idt = edge_index.dtype
    per = e // NW
    padt = nchunk * CH - per
    fill = jnp.arange(padt, dtype=idt)
    src = jnp.concatenate(
        [edge_index[0].reshape(NW, per),
         jnp.broadcast_to(fill % n, (NW, padt))],
        axis=1).reshape(NW, nchunk, CH)
    dst = jnp.concatenate(
        [edge_index[1].reshape(NW, per),
         jnp.broadcast_to(n + fill % TRASH, (NW, padt))],
        axis=1).reshape(NW, nchunk, CH)
    zeros = jnp.zeros((n, hdim), jnp.float32)
    parts = _make_sc_segment_sum(n, hdim, nchunk)(h, src, dst, zeros)

    return _final(parts, h, W_msg, b_msg.reshape(1, hdim), W_out1,
                  b_out1.reshape(1, hdim), W_out2, b_out2.reshape(1, 1),
                  block_rows=1000)


# speedup vs baseline: 1.0694x; 1.0694x over previous
import functools

import jax
import jax.numpy as jnp
from jax import lax
from jax.experimental import pallas as pl
from jax.experimental.pallas import tpu as pltpu
from jax.experimental.pallas import tpu_sc as plsc

NC = 2
NS = 16
NW = NC * NS
CH = 128
TRASH = 8


def _encode_body(x_ref, w_ref, b_ref, o_ref):
    acc = jnp.dot(x_ref[...], w_ref[...], preferred_element_type=jnp.float32)
    o_ref[...] = jnp.maximum(acc + b_ref[...], 0.0)


def _encode(x, w, b2d, block_rows):
    n, d = x.shape
    h = w.shape[1]
    grid = n // block_rows
    return pl.pallas_call(
        _encode_body,
        grid=(grid,),
        in_specs=[
            pl.BlockSpec((block_rows, d), lambda i: (i, 0)),
            pl.BlockSpec((d, h), lambda i: (0, 0)),
            pl.BlockSpec((1, h), lambda i: (0, 0)),
        ],
        out_specs=pl.BlockSpec((block_rows, h), lambda i: (i, 0)),
        out_shape=jax.ShapeDtypeStruct((n, h), jnp.float32),
    )(x, w, b2d)


def _make_sc_segment_sum(n, hdim, nchunk):
    base = (n // NS) // 8 * 8
    rem = n - NS * base
    step = 40
    passes = tuple((i, min(step, nchunk - i)) for i in range(0, nchunk, step))
    idx_rows = step
    mesh = plsc.VectorSubcoreMesh(core_axis_name="c", subcore_axis_name="s")

    @functools.partial(
        pl.kernel,
        mesh=mesh,
        out_type=jax.ShapeDtypeStruct((NC, n, hdim), jnp.float32),
        scratch_types=[
            pltpu.VMEM((idx_rows, CH), jnp.int32),
            pltpu.VMEM((idx_rows, CH), jnp.int32),
            pltpu.VMEM((CH, hdim), jnp.float32),
            pltpu.VMEM((CH, hdim), jnp.float32),
            pltpu.VMEM_SHARED((n + TRASH, hdim), jnp.float32),
            pltpu.SemaphoreType.DMA,
            pltpu.SemaphoreType.DMA,
            pltpu.SemaphoreType.DMA,
            pltpu.SemaphoreType.DMA,
        ],
    )
    def sc_seg_sum(h_hbm, src_hbm, dst_hbm, zeros_hbm, out_hbm,
                   src_v, dst_v, rows0_v, rows1_v, agg_sh, sem0, sem1,
                   ssem0, ssem1):
        c = lax.axis_index("c")
        s = lax.axis_index("s")
        wid = s * NC + c
        row0 = s * base
        pltpu.sync_copy(zeros_hbm.at[pl.ds(row0, base)],
                        agg_sh.at[pl.ds(row0, base)])

        @pl.when(s == NS - 1)
        def _zero_tail():
            pltpu.sync_copy(zeros_hbm.at[pl.ds(NS * base, rem)],
                            agg_sh.at[pl.ds(NS * base, rem)])
        plsc.subcore_barrier()

        bufs = (rows0_v, rows1_v)
        gsems = (sem0, sem1)
        ssems = (ssem0, ssem1)
        nbuf = 2

        for start, cnt in passes:
            pltpu.sync_copy(src_hbm.at[wid, pl.ds(start, cnt)],
                            src_v.at[pl.ds(0, cnt)])
            pltpu.sync_copy(dst_hbm.at[wid, pl.ds(start, cnt)],
                            dst_v.at[pl.ds(0, cnt)])
            for b in range(min(nbuf, cnt)):
                pltpu.async_copy(h_hbm.at[src_v.at[b]], bufs[b], gsems[b])

            def body(t, carry):
                j0 = t * nbuf
                for b in range(nbuf):
                    j = j0 + b
                    pltpu.make_async_copy(h_hbm.at[src_v.at[j]], bufs[b],
                                          gsems[b]).wait()
                    pltpu.async_copy(bufs[b], agg_sh.at[dst_v.at[j]],
                                     ssems[b], add=True)

                    @pl.when(j + nbuf < cnt)
                    def _next():
                        pltpu.make_async_copy(bufs[b],
                                              agg_sh.at[dst_v.at[j]],
                                              ssems[b]).wait()
                        pltpu.async_copy(h_hbm.at[src_v.at[j + nbuf]],
                                         bufs[b], gsems[b])
                return carry

            lax.fori_loop(0, cnt // nbuf, body, 0)
            for j in range(cnt // nbuf * nbuf, cnt):
                b = j % nbuf
                pltpu.make_async_copy(h_hbm.at[src_v.at[j]], bufs[b],
                                      gsems[b]).wait()
                pltpu.async_copy(bufs[b], agg_sh.at[dst_v.at[j]],
                                 ssems[b], add=True)
            for j in range(max(0, cnt - nbuf), cnt):
                b = j % nbuf
                pltpu.make_async_copy(bufs[b], agg_sh.at[dst_v.at[j]],
                                      ssems[b]).wait()
        plsc.subcore_barrier()
        pltpu.sync_copy(agg_sh.at[pl.ds(row0, base)],
                        out_hbm.at[c, pl.ds(row0, base)])

        @pl.when(s == NS - 1)
        def _out_tail():
            pltpu.sync_copy(agg_sh.at[pl.ds(NS * base, rem)],
                            out_hbm.at[c, pl.ds(NS * base, rem)])

    return sc_seg_sum


def _final_body(p_ref, h_ref, wm_ref, bm_ref, w1_ref, b1_ref, w2_ref, b2_ref,
                o_ref):
    agg = p_ref[0] + p_ref[1]
    t = jnp.dot(agg, wm_ref[...], preferred_element_type=jnp.float32)
    t = jnp.maximum(t + bm_ref[...], 0.0) + h_ref[...]
    hid = jnp.dot(t, w1_ref[...], preferred_element_type=jnp.float32)
    hid = jnp.maximum(hid + b1_ref[...], 0.0)
    o_ref[...] = jnp.dot(hid, w2_ref[...],
                         preferred_element_type=jnp.float32) + b2_ref[...]


def _final(parts, h, wm, bm2d, w1, b12d, w2, b22d, block_rows):
    n, hdim = h.shape
    grid = n // block_rows
    return pl.pallas_call(
        _final_body,
        grid=(grid,),
        in_specs=[
            pl.BlockSpec((NC, block_rows, hdim), lambda i: (0, i, 0)),
            pl.BlockSpec((block_rows, hdim), lambda i: (i, 0)),
            pl.BlockSpec((hdim, hdim), lambda i: (0, 0)),
            pl.BlockSpec((1, hdim), lambda i: (0, 0)),
            pl.BlockSpec((hdim, hdim), lambda i: (0, 0)),
            pl.BlockSpec((1, hdim), lambda i: (0, 0)),
            pl.BlockSpec((hdim, 1), lambda i: (0, 0)),
            pl.BlockSpec((1, 1), lambda i: (0, 0)),
        ],
        out_specs=pl.BlockSpec((block_rows, 1), lambda i: (i, 0)),
        out_shape=jax.ShapeDtypeStruct((n, 1), jnp.float32),
    )(parts, h, wm, bm2d, w1, b12d, w2, b22d)


def kernel(x, edge_index, W_enc, b_enc, W_msg, b_msg, W_out1, b_out1,
           W_out2, b_out2):
    n, d = x.shape
    hdim = W_enc.shape[1]
    e = edge_index.shape[1]
    nchunk = -(-e // (NW * CH))

    h = _encode(x, W_enc, b_enc.reshape(1, hdim), block_rows=1000)

    idt = edge_index.dtype
    pad = NW * nchunk * CH - e
    fill = jnp.arange(pad, dtype=idt)
    src = jnp.concatenate(
        [edge_index[0], fill % n]).reshape(NW, nchunk, CH)
    dst = jnp.concatenate(
        [edge_index[1], n + fill % TRASH]).reshape(NW, nchunk, CH)
    zeros = jnp.zeros((n, hdim), jnp.float32)
    parts = _make_sc_segment_sum(n, hdim, nchunk)(h, src, dst, zeros)

    return _final(parts, h, W_msg, b_msg.reshape(1, hdim), W_out1,
                  b_out1.reshape(1, hdim), W_out2, b_out2.reshape(1, 1),
                  block_rows=1000)


# speedup vs baseline: 1.0882x; 1.0175x over previous
import functools

import jax
import jax.numpy as jnp
from jax import lax
from jax.experimental import pallas as pl
from jax.experimental.pallas import tpu as pltpu
from jax.experimental.pallas import tpu_sc as plsc

NC = 2
NS = 16
NW = NC * NS
CH = 128
STEP = 24
TRASH = 8


def _encode_body(x_ref, w_ref, b_ref, o_ref):
    acc = jnp.dot(x_ref[...], w_ref[...], preferred_element_type=jnp.float32)
    o_ref[...] = jnp.maximum(acc + b_ref[...], 0.0)


def _encode(x, w, b2d, block_rows):
    n, d = x.shape
    h = w.shape[1]
    grid = n // block_rows
    return pl.pallas_call(
        _encode_body,
        grid=(grid,),
        in_specs=[
            pl.BlockSpec((block_rows, d), lambda i: (i, 0)),
            pl.BlockSpec((d, h), lambda i: (0, 0)),
            pl.BlockSpec((1, h), lambda i: (0, 0)),
        ],
        out_specs=pl.BlockSpec((block_rows, h), lambda i: (i, 0)),
        out_shape=jax.ShapeDtypeStruct((n, h), jnp.float32),
    )(x, w, b2d)


def _make_sc_segment_sum(n, hdim, nchunk):
    base = (n // NS) // 8 * 8
    rem = n - NS * base
    npass = -(-nchunk // STEP)
    mesh = plsc.VectorSubcoreMesh(core_axis_name="c", subcore_axis_name="s")

    @functools.partial(
        pl.kernel,
        mesh=mesh,
        out_type=jax.ShapeDtypeStruct((NC, n, hdim), jnp.float32),
        scratch_types=[
            pltpu.VMEM((2, STEP, CH), jnp.int32),
            pltpu.VMEM((2, STEP, CH), jnp.int32),
            pltpu.VMEM((CH, hdim), jnp.float32),
            pltpu.VMEM((CH, hdim), jnp.float32),
            pltpu.VMEM_SHARED((n + TRASH, hdim), jnp.float32),
            pltpu.SemaphoreType.DMA,
            pltpu.SemaphoreType.DMA,
            pltpu.SemaphoreType.DMA,
            pltpu.SemaphoreType.DMA,
            pltpu.SemaphoreType.DMA,
            pltpu.SemaphoreType.DMA,
        ],
    )
    def sc_seg_sum(h_hbm, src_hbm, dst_hbm, zeros_hbm, out_hbm,
                   src_v, dst_v, rows0_v, rows1_v, agg_sh, sem0, sem1,
                   ssem0, ssem1, isem_s, isem_d):
        c = lax.axis_index("c")
        s = lax.axis_index("s")
        wid = s * NC + c
        row0 = s * base
        pltpu.async_copy(src_hbm.at[wid, pl.ds(0, STEP)], src_v.at[0],
                         isem_s)
        pltpu.async_copy(dst_hbm.at[wid, pl.ds(0, STEP)], dst_v.at[0],
                         isem_d)
        pltpu.sync_copy(zeros_hbm.at[pl.ds(row0, base)],
                        agg_sh.at[pl.ds(row0, base)])

        @pl.when(s == NS - 1)
        def _zero_tail():
            pltpu.sync_copy(zeros_hbm.at[pl.ds(NS * base, rem)],
                            agg_sh.at[pl.ds(NS * base, rem)])
        plsc.subcore_barrier()
        pltpu.make_async_copy(src_hbm.at[wid, pl.ds(0, STEP)], src_v.at[0],
                              isem_s).wait()
        pltpu.make_async_copy(dst_hbm.at[wid, pl.ds(0, STEP)], dst_v.at[0],
                              isem_d).wait()

        bufs = (rows0_v, rows1_v)
        gsems = (sem0, sem1)
        ssems = (ssem0, ssem1)
        nbuf = 2
        for b in range(nbuf):
            pltpu.async_copy(h_hbm.at[src_v.at[0, b]], bufs[b], gsems[b])

        def body(t, carry):
            j0 = t * nbuf
            for b in range(nbuf):
                j = j0 + b
                p = j // STEP
                sl = lax.rem(p, 2)
                jo = lax.rem(j, STEP)

                @pl.when(jnp.logical_and(jo == 0, p + 1 < npass))
                def _prefetch():
                    q = p + 1
                    sq = lax.rem(q, 2)
                    pltpu.async_copy(src_hbm.at[wid, pl.ds(q * STEP, STEP)],
                                     src_v.at[sq], isem_s)
                    pltpu.async_copy(dst_hbm.at[wid, pl.ds(q * STEP, STEP)],
                                     dst_v.at[sq], isem_d)

                pltpu.make_async_copy(h_hbm.at[src_v.at[sl, jo]], bufs[b],
                                      gsems[b]).wait()
                pltpu.async_copy(bufs[b], agg_sh.at[dst_v.at[sl, jo]],
                                 ssems[b], add=True)
                jn = j + nbuf

                @pl.when(jn < nchunk)
                def _next():
                    pltpu.make_async_copy(bufs[b],
                                          agg_sh.at[dst_v.at[sl, jo]],
                                          ssems[b]).wait()
                    pn = jn // STEP
                    sn = lax.rem(pn, 2)
                    jno = lax.rem(jn, STEP)

                    @pl.when(jno == 0)
                    def _wait_idx():
                        pltpu.make_async_copy(
                            src_hbm.at[wid, pl.ds(pn * STEP, STEP)],
                            src_v.at[sn], isem_s).wait()
                        pltpu.make_async_copy(
                            dst_hbm.at[wid, pl.ds(pn * STEP, STEP)],
                            dst_v.at[sn], isem_d).wait()
                    pltpu.async_copy(h_hbm.at[src_v.at[sn, jno]],
                                     bufs[b], gsems[b])
            return carry

        lax.fori_loop(0, nchunk // nbuf, body, 0)
        for j in range(nchunk // nbuf * nbuf, nchunk):
            b = j % nbuf
            sl = (j // STEP) % 2
            pltpu.make_async_copy(h_hbm.at[src_v.at[sl, j % STEP]], bufs[b],
                                  gsems[b]).wait()
            pltpu.async_copy(bufs[b], agg_sh.at[dst_v.at[sl, j % STEP]],
                             ssems[b], add=True)
        for j in range(max(0, nchunk - nbuf), nchunk):
            b = j % nbuf
            sl = (j // STEP) % 2
            pltpu.make_async_copy(bufs[b], agg_sh.at[dst_v.at[sl, j % STEP]],
                                  ssems[b]).wait()
        plsc.subcore_barrier()
        pltpu.sync_copy(agg_sh.at[pl.ds(row0, base)],
                        out_hbm.at[c, pl.ds(row0, base)])

        @pl.when(s == NS - 1)
        def _out_tail():
            pltpu.sync_copy(agg_sh.at[pl.ds(NS * base, rem)],
                            out_hbm.at[c, pl.ds(NS * base, rem)])

    return sc_seg_sum


def _final_body(p_ref, h_ref, wm_ref, bm_ref, w1_ref, b1_ref, w2_ref, b2_ref,
                o_ref):
    agg = p_ref[0] + p_ref[1]
    t = jnp.dot(agg, wm_ref[...], preferred_element_type=jnp.float32)
    t = jnp.maximum(t + bm_ref[...], 0.0) + h_ref[...]
    hid = jnp.dot(t, w1_ref[...], preferred_element_type=jnp.float32)
    hid = jnp.maximum(hid + b1_ref[...], 0.0)
    o_ref[...] = jnp.dot(hid, w2_ref[...],
                         preferred_element_type=jnp.float32) + b2_ref[...]


def _final(parts, h, wm, bm2d, w1, b12d, w2, b22d, block_rows):
    n, hdim = h.shape
    grid = n // block_rows
    return pl.pallas_call(
        _final_body,
        grid=(grid,),
        in_specs=[
            pl.BlockSpec((NC, block_rows, hdim), lambda i: (0, i, 0)),
            pl.BlockSpec((block_rows, hdim), lambda i: (i, 0)),
            pl.BlockSpec((hdim, hdim), lambda i: (0, 0)),
            pl.BlockSpec((1, hdim), lambda i: (0, 0)),
            pl.BlockSpec((hdim, hdim), lambda i: (0, 0)),
            pl.BlockSpec((1, hdim), lambda i: (0, 0)),
            pl.BlockSpec((hdim, 1), lambda i: (0, 0)),
            pl.BlockSpec((1, 1), lambda i: (0, 0)),
        ],
        out_specs=pl.BlockSpec((block_rows, 1), lambda i: (i, 0)),
        out_shape=jax.ShapeDtypeStruct((n, 1), jnp.float32),
    )(parts, h, wm, bm2d, w1, b12d, w2, b22d)


def kernel(x, edge_index, W_enc, b_enc, W_msg, b_msg, W_out1, b_out1,
           W_out2, b_out2):
    n, d = x.shape
    hdim = W_enc.shape[1]
    e = edge_index.shape[1]
    nchunk = -(-e // (NW * CH))

    h = _encode(x, W_enc, b_enc.reshape(1, hdim), block_rows=1000)

    idt = edge_index.dtype
    pad = NW * nchunk * CH - e
    fill = jnp.arange(pad, dtype=idt)
    src = jnp.concatenate(
        [edge_index[0], fill % n]).reshape(NW, nchunk, CH)
    dst = jnp.concatenate(
        [edge_index[1], n + fill % TRASH]).reshape(NW, nchunk, CH)
    cpad = -(-nchunk // STEP) * STEP - nchunk
    src = jnp.pad(src, ((0, 0), (0, cpad), (0, 0)))
    dst = jnp.pad(dst, ((0, 0), (0, cpad), (0, 0)))
    zeros = jnp.zeros((n, hdim), jnp.float32)
    parts = _make_sc_segment_sum(n, hdim, nchunk)(h, src, dst, zeros)

    return _final(parts, h, W_msg, b_msg.reshape(1, hdim), W_out1,
                  b_out1.reshape(1, hdim), W_out2, b_out2.reshape(1, 1),
                  block_rows=1000)


# speedup vs baseline: 1.1134x; 1.0232x over previous
import functools

import jax
import jax.numpy as jnp
from jax import lax
from jax.experimental import pallas as pl
from jax.experimental.pallas import tpu as pltpu
from jax.experimental.pallas import tpu_sc as plsc

NC = 2
NS = 16
NW = NC * NS
CH = 80
NB = 3
STEP = 24
TRASH = 8


def _encode_body(x_ref, w_ref, b_ref, o_ref):
    acc = jnp.dot(x_ref[...], w_ref[...], preferred_element_type=jnp.float32)
    o_ref[...] = jnp.maximum(acc + b_ref[...], 0.0)


def _encode(x, w, b2d, block_rows):
    n, d = x.shape
    h = w.shape[1]
    grid = n // block_rows
    return pl.pallas_call(
        _encode_body,
        grid=(grid,),
        in_specs=[
            pl.BlockSpec((block_rows, d), lambda i: (i, 0)),
            pl.BlockSpec((d, h), lambda i: (0, 0)),
            pl.BlockSpec((1, h), lambda i: (0, 0)),
        ],
        out_specs=pl.BlockSpec((block_rows, h), lambda i: (i, 0)),
        out_shape=jax.ShapeDtypeStruct((n, h), jnp.float32),
    )(x, w, b2d)


def _make_sc_segment_sum(n, hdim, nchunk):
    base = (n // NS) // 8 * 8
    rem = n - NS * base
    npass = -(-nchunk // STEP)
    mesh = plsc.VectorSubcoreMesh(core_axis_name="c", subcore_axis_name="s")

    @functools.partial(
        pl.kernel,
        mesh=mesh,
        out_type=jax.ShapeDtypeStruct((NC, n, hdim), jnp.float32),
        scratch_types=[
            pltpu.VMEM((2, STEP, CH), jnp.int32),
            pltpu.VMEM((2, STEP, CH), jnp.int32),
            pltpu.VMEM((CH, hdim), jnp.float32),
            pltpu.VMEM((CH, hdim), jnp.float32),
            pltpu.VMEM((CH, hdim), jnp.float32),
            pltpu.VMEM_SHARED((n + TRASH, hdim), jnp.float32),
            pltpu.SemaphoreType.DMA,
            pltpu.SemaphoreType.DMA,
            pltpu.SemaphoreType.DMA,
            pltpu.SemaphoreType.DMA,
            pltpu.SemaphoreType.DMA,
            pltpu.SemaphoreType.DMA,
            pltpu.SemaphoreType.DMA,
            pltpu.SemaphoreType.DMA,
        ],
    )
    def sc_seg_sum(h_hbm, src_hbm, dst_hbm, zeros_hbm, out_hbm,
                   src_v, dst_v, rows0_v, rows1_v, rows2_v, agg_sh,
                   sem0, sem1, sem2, ssem0, ssem1, ssem2, isem_s, isem_d):
        c = lax.axis_index("c")
        s = lax.axis_index("s")
        wid = s * NC + c
        row0 = s * base
        pltpu.async_copy(src_hbm.at[wid, pl.ds(0, STEP)], src_v.at[0],
                         isem_s)
        pltpu.async_copy(dst_hbm.at[wid, pl.ds(0, STEP)], dst_v.at[0],
                         isem_d)
        pltpu.sync_copy(zeros_hbm.at[pl.ds(row0, base)],
                        agg_sh.at[pl.ds(row0, base)])

        @pl.when(s == NS - 1)
        def _zero_tail():
            pltpu.sync_copy(zeros_hbm.at[pl.ds(NS * base, rem)],
                            agg_sh.at[pl.ds(NS * base, rem)])
        plsc.subcore_barrier()
        pltpu.make_async_copy(src_hbm.at[wid, pl.ds(0, STEP)], src_v.at[0],
                              isem_s).wait()
        pltpu.make_async_copy(dst_hbm.at[wid, pl.ds(0, STEP)], dst_v.at[0],
                              isem_d).wait()

        bufs = (rows0_v, rows1_v, rows2_v)
        gsems = (sem0, sem1, sem2)
        ssems = (ssem0, ssem1, ssem2)
        for b in range(NB - 1):
            pltpu.async_copy(h_hbm.at[src_v.at[0, b]], bufs[b], gsems[b])

        def body(t, carry):
            j0 = t * NB
            for b in range(NB):
                j = j0 + b
                p = j // STEP
                sl = lax.rem(p, 2)
                jo = lax.rem(j, STEP)

                pltpu.make_async_copy(h_hbm.at[src_v.at[sl, jo]], bufs[b],
                                      gsems[b]).wait()
                pltpu.async_copy(bufs[b], agg_sh.at[dst_v.at[sl, jo]],
                                 ssems[b], add=True)
                jp = j - 1
                bp = (b - 1) % NB

                @pl.when(jp >= 0)
                def _wait_prev_scatter():
                    pp = jp // STEP
                    slp = lax.rem(pp, 2)
                    jpo = lax.rem(jp, STEP)
                    pltpu.make_async_copy(bufs[bp],
                                          agg_sh.at[dst_v.at[slp, jpo]],
                                          ssems[bp]).wait()

                @pl.when(jnp.logical_and(jo == 0, p + 1 < npass))
                def _prefetch():
                    q = p + 1
                    sq = lax.rem(q, 2)
                    pltpu.async_copy(src_hbm.at[wid, pl.ds(q * STEP, STEP)],
                                     src_v.at[sq], isem_s)
                    pltpu.async_copy(dst_hbm.at[wid, pl.ds(q * STEP, STEP)],
                                     dst_v.at[sq], isem_d)

                jn = j + NB - 1

                @pl.when(jn < nchunk)
                def _next():
                    pn = jn // STEP
                    sn = lax.rem(pn, 2)
                    jno = lax.rem(jn, STEP)

                    @pl.when(jno == 0)
                    def _wait_idx():
                        pltpu.make_async_copy(
                            src_hbm.at[wid, pl.ds(pn * STEP, STEP)],
                            src_v.at[sn], isem_s).wait()
                        pltpu.make_async_copy(
                            dst_hbm.at[wid, pl.ds(pn * STEP, STEP)],
                            dst_v.at[sn], isem_d).wait()
                    pltpu.async_copy(h_hbm.at[src_v.at[sn, jno]],
                                     bufs[bp], gsems[bp])
            return carry

        lax.fori_loop(0, nchunk // NB, body, 0)
        for j in range(nchunk // NB * NB, nchunk):
            b = j % NB
            sl = (j // STEP) % 2
            pltpu.make_async_copy(h_hbm.at[src_v.at[sl, j % STEP]], bufs[b],
                                  gsems[b]).wait()
            pltpu.async_copy(bufs[b], agg_sh.at[dst_v.at[sl, j % STEP]],
                             ssems[b], add=True)
            jp = j - 1
            if jp >= 0:
                bp = jp % NB
                slp = (jp // STEP) % 2
                pltpu.make_async_copy(bufs[bp],
                                      agg_sh.at[dst_v.at[slp, jp % STEP]],
                                      ssems[bp]).wait()
        j = nchunk - 1
        pltpu.make_async_copy(bufs[j % NB],
                              agg_sh.at[dst_v.at[(j // STEP) % 2,
                                                 j % STEP]],
                              ssems[j % NB]).wait()
        plsc.subcore_barrier()
        pltpu.sync_copy(agg_sh.at[pl.ds(row0, base)],
                        out_hbm.at[c, pl.ds(row0, base)])

        @pl.when(s == NS - 1)
        def _out_tail():
            pltpu.sync_copy(agg_sh.at[pl.ds(NS * base, rem)],
                            out_hbm.at[c, pl.ds(NS * base, rem)])

    return sc_seg_sum


def _final_body(p_ref, h_ref, wm_ref, bm_ref, w1_ref, b1_ref, w2_ref, b2_ref,
                o_ref):
    agg = p_ref[0] + p_ref[1]
    t = jnp.dot(agg, wm_ref[...], preferred_element_type=jnp.float32)
    t = jnp.maximum(t + bm_ref[...], 0.0) + h_ref[...]
    hid = jnp.dot(t, w1_ref[...], preferred_element_type=jnp.float32)
    hid = jnp.maximum(hid + b1_ref[...], 0.0)
    o_ref[...] = jnp.dot(hid, w2_ref[...],
                         preferred_element_type=jnp.float32) + b2_ref[...]


def _final(parts, h, wm, bm2d, w1, b12d, w2, b22d, block_rows):
    n, hdim = h.shape
    grid = n // block_rows
    return pl.pallas_call(
        _final_body,
        grid=(grid,),
        in_specs=[
            pl.BlockSpec((NC, block_rows, hdim), lambda i: (0, i, 0)),
            pl.BlockSpec((block_rows, hdim), lambda i: (i, 0)),
            pl.BlockSpec((hdim, hdim), lambda i: (0, 0)),
            pl.BlockSpec((1, hdim), lambda i: (0, 0)),
            pl.BlockSpec((hdim, hdim), lambda i: (0, 0)),
            pl.BlockSpec((1, hdim), lambda i: (0, 0)),
            pl.BlockSpec((hdim, 1), lambda i: (0, 0)),
            pl.BlockSpec((1, 1), lambda i: (0, 0)),
        ],
        out_specs=pl.BlockSpec((block_rows, 1), lambda i: (i, 0)),
        out_shape=jax.ShapeDtypeStruct((n, 1), jnp.float32),
    )(parts, h, wm, bm2d, w1, b12d, w2, b22d)


def kernel(x, edge_index, W_enc, b_enc, W_msg, b_msg, W_out1, b_out1,
           W_out2, b_out2):
    n, d = x.shape
    hdim = W_enc.shape[1]
    e = edge_index.shape[1]
    nchunk = -(-e // (NW * CH))

    h = _encode(x, W_enc, b_enc.reshape(1, hdim), block_rows=1000)

    idt = edge_index.dtype
    pad = NW * nchunk * CH - e
    fill = jnp.arange(pad, dtype=idt)
    src = jnp.concatenate(
        [edge_index[0], fill % n]).reshape(NW, nchunk, CH)
    dst = jnp.concatenate(
        [edge_index[1], n + fill % TRASH]).reshape(NW, nchunk, CH)
    cpad = -(-nchunk // STEP) * STEP - nchunk
    src = jnp.pad(src, ((0, 0), (0, cpad), (0, 0)))
    dst = jnp.pad(dst, ((0, 0), (0, cpad), (0, 0)))
    zeros = jnp.zeros((n, hdim), jnp.float32)
    parts = _make_sc_segment_sum(n, hdim, nchunk)(h, src, dst, zeros)

    return _final(parts, h, W_msg, b_msg.reshape(1, hdim), W_out1,
                  b_out1.reshape(1, hdim), W_out2, b_out2.reshape(1, 1),
                  block_rows=1000)


# speedup vs baseline: 1.1200x; 1.0059x over previous
import functools

import jax
import jax.numpy as jnp
from jax import lax
from jax.experimental import pallas as pl
from jax.experimental.pallas import tpu as pltpu
from jax.experimental.pallas import tpu_sc as plsc

NC = 2
NS = 16
NW = NC * NS
CH = 96
NB = 3
STEP = 16
TRASH = 8


def _encode_body(x_ref, w_ref, b_ref, o_ref):
    acc = jnp.dot(x_ref[...], w_ref[...], preferred_element_type=jnp.float32)
    o_ref[...] = jnp.maximum(acc + b_ref[...], 0.0)


def _encode(x, w, b2d, block_rows):
    n, d = x.shape
    h = w.shape[1]
    grid = n // block_rows
    return pl.pallas_call(
        _encode_body,
        grid=(grid,),
        in_specs=[
            pl.BlockSpec((block_rows, d), lambda i: (i, 0)),
            pl.BlockSpec((d, h), lambda i: (0, 0)),
            pl.BlockSpec((1, h), lambda i: (0, 0)),
        ],
        out_specs=pl.BlockSpec((block_rows, h), lambda i: (i, 0)),
        out_shape=jax.ShapeDtypeStruct((n, h), jnp.float32),
    )(x, w, b2d)


def _make_sc_segment_sum(n, hdim, nchunk):
    base = (n // NS) // 8 * 8
    rem = n - NS * base
    npass = -(-nchunk // STEP)
    mesh = plsc.VectorSubcoreMesh(core_axis_name="c", subcore_axis_name="s")

    @functools.partial(
        pl.kernel,
        mesh=mesh,
        out_type=jax.ShapeDtypeStruct((NC, n, hdim), jnp.float32),
        scratch_types=[
            pltpu.VMEM((2, STEP, CH), jnp.int32),
            pltpu.VMEM((2, STEP, CH), jnp.int32),
            pltpu.VMEM((CH, hdim), jnp.float32),
            pltpu.VMEM((CH, hdim), jnp.float32),
            pltpu.VMEM((CH, hdim), jnp.float32),
            pltpu.VMEM_SHARED((n + TRASH, hdim), jnp.float32),
            pltpu.SemaphoreType.DMA,
            pltpu.SemaphoreType.DMA,
            pltpu.SemaphoreType.DMA,
            pltpu.SemaphoreType.DMA,
            pltpu.SemaphoreType.DMA,
            pltpu.SemaphoreType.DMA,
            pltpu.SemaphoreType.DMA,
            pltpu.SemaphoreType.DMA,
        ],
    )
    def sc_seg_sum(h_hbm, src_hbm, dst_hbm, zeros_hbm, out_hbm,
                   src_v, dst_v, rows0_v, rows1_v, rows2_v, agg_sh,
                   sem0, sem1, sem2, ssem0, ssem1, ssem2, isem_s, isem_d):
        c = lax.axis_index("c")
        s = lax.axis_index("s")
        wid = s * NC + c
        row0 = s * base
        pltpu.async_copy(src_hbm.at[wid, pl.ds(0, STEP)], src_v.at[0],
                         isem_s)
        pltpu.async_copy(dst_hbm.at[wid, pl.ds(0, STEP)], dst_v.at[0],
                         isem_d)
        pltpu.sync_copy(zeros_hbm.at[pl.ds(row0, base)],
                        agg_sh.at[pl.ds(row0, base)])

        @pl.when(s == NS - 1)
        def _zero_tail():
            pltpu.sync_copy(zeros_hbm.at[pl.ds(NS * base, rem)],
                            agg_sh.at[pl.ds(NS * base, rem)])
        plsc.subcore_barrier()
        pltpu.make_async_copy(src_hbm.at[wid, pl.ds(0, STEP)], src_v.at[0],
                              isem_s).wait()
        pltpu.make_async_copy(dst_hbm.at[wid, pl.ds(0, STEP)], dst_v.at[0],
                              isem_d).wait()

        bufs = (rows0_v, rows1_v, rows2_v)
        gsems = (sem0, sem1, sem2)
        ssems = (ssem0, ssem1, ssem2)
        for b in range(NB - 1):
            pltpu.async_copy(h_hbm.at[src_v.at[0, b]], bufs[b], gsems[b])

        def body(t, carry):
            j0 = t * NB
            for b in range(NB):
                j = j0 + b
                p = j // STEP
                sl = lax.rem(p, 2)
                jo = lax.rem(j, STEP)

                pltpu.make_async_copy(h_hbm.at[src_v.at[sl, jo]], bufs[b],
                                      gsems[b]).wait()
                pltpu.async_copy(bufs[b], agg_sh.at[dst_v.at[sl, jo]],
                                 ssems[b], add=True)
                jp = j - 1
                bp = (b - 1) % NB

                @pl.when(jp >= 0)
                def _wait_prev_scatter():
                    pp = jp // STEP
                    slp = lax.rem(pp, 2)
                    jpo = lax.rem(jp, STEP)
                    pltpu.make_async_copy(bufs[bp],
                                          agg_sh.at[dst_v.at[slp, jpo]],
                                          ssems[bp]).wait()

                @pl.when(jnp.logical_and(jo == 0, p + 1 < npass))
                def _prefetch():
                    q = p + 1
                    sq = lax.rem(q, 2)
                    pltpu.async_copy(src_hbm.at[wid, pl.ds(q * STEP, STEP)],
                                     src_v.at[sq], isem_s)
                    pltpu.async_copy(dst_hbm.at[wid, pl.ds(q * STEP, STEP)],
                                     dst_v.at[sq], isem_d)

                jn = j + NB - 1

                @pl.when(jn < nchunk)
                def _next():
                    pn = jn // STEP
                    sn = lax.rem(pn, 2)
                    jno = lax.rem(jn, STEP)

                    @pl.when(jno == 0)
                    def _wait_idx():
                        pltpu.make_async_copy(
                            src_hbm.at[wid, pl.ds(pn * STEP, STEP)],
                            src_v.at[sn], isem_s).wait()
                        pltpu.make_async_copy(
                            dst_hbm.at[wid, pl.ds(pn * STEP, STEP)],
                            dst_v.at[sn], isem_d).wait()
                    pltpu.async_copy(h_hbm.at[src_v.at[sn, jno]],
                                     bufs[bp], gsems[bp])
            return carry

        lax.fori_loop(0, nchunk // NB, body, 0)
        for j in range(nchunk // NB * NB, nchunk):
            b = j % NB
            sl = (j // STEP) % 2
            pltpu.make_async_copy(h_hbm.at[src_v.at[sl, j % STEP]], bufs[b],
                                  gsems[b]).wait()
            pltpu.async_copy(bufs[b], agg_sh.at[dst_v.at[sl, j % STEP]],
                             ssems[b], add=True)
            jp = j - 1
            if jp >= 0:
                bp = jp % NB
                slp = (jp // STEP) % 2
                pltpu.make_async_copy(bufs[bp],
                                      agg_sh.at[dst_v.at[slp, jp % STEP]],
                                      ssems[bp]).wait()
        j = nchunk - 1
        pltpu.make_async_copy(bufs[j % NB],
                              agg_sh.at[dst_v.at[(j // STEP) % 2,
                                                 j % STEP]],
                              ssems[j % NB]).wait()
        plsc.subcore_barrier()
        pltpu.sync_copy(agg_sh.at[pl.ds(row0, base)],
                        out_hbm.at[c, pl.ds(row0, base)])

        @pl.when(s == NS - 1)
        def _out_tail():
            pltpu.sync_copy(agg_sh.at[pl.ds(NS * base, rem)],
                            out_hbm.at[c, pl.ds(NS * base, rem)])

    return sc_seg_sum


def _final_body(p_ref, h_ref, wm_ref, bm_ref, w1_ref, b1_ref, w2_ref, b2_ref,
                o_ref):
    agg = p_ref[0] + p_ref[1]
    t = jnp.dot(agg, wm_ref[...], preferred_element_type=jnp.float32)
    t = jnp.maximum(t + bm_ref[...], 0.0) + h_ref[...]
    hid = jnp.dot(t, w1_ref[...], preferred_element_type=jnp.float32)
    hid = jnp.maximum(hid + b1_ref[...], 0.0)
    o_ref[...] = jnp.dot(hid, w2_ref[...],
                         preferred_element_type=jnp.float32) + b2_ref[...]


def _final(parts, h, wm, bm2d, w1, b12d, w2, b22d, block_rows):
    n, hdim = h.shape
    grid = n // block_rows
    return pl.pallas_call(
        _final_body,
        grid=(grid,),
        in_specs=[
            pl.BlockSpec((NC, block_rows, hdim), lambda i: (0, i, 0)),
            pl.BlockSpec((block_rows, hdim), lambda i: (i, 0)),
            pl.BlockSpec((hdim, hdim), lambda i: (0, 0)),
            pl.BlockSpec((1, hdim), lambda i: (0, 0)),
            pl.BlockSpec((hdim, hdim), lambda i: (0, 0)),
            pl.BlockSpec((1, hdim), lambda i: (0, 0)),
            pl.BlockSpec((hdim, 1), lambda i: (0, 0)),
            pl.BlockSpec((1, 1), lambda i: (0, 0)),
        ],
        out_specs=pl.BlockSpec((block_rows, 1), lambda i: (i, 0)),
        out_shape=jax.ShapeDtypeStruct((n, 1), jnp.float32),
    )(parts, h, wm, bm2d, w1, b12d, w2, b22d)


def kernel(x, edge_index, W_enc, b_enc, W_msg, b_msg, W_out1, b_out1,
           W_out2, b_out2):
    n, d = x.shape
    hdim = W_enc.shape[1]
    e = edge_index.shape[1]
    nchunk = -(-e // (NW * CH))

    h = _encode(x, W_enc, b_enc.reshape(1, hdim), block_rows=1000)

    idt = edge_index.dtype
    pad = NW * nchunk * CH - e
    fill = jnp.arange(pad, dtype=idt)
    src = jnp.concatenate(
        [edge_index[0], fill % n]).reshape(NW, nchunk, CH)
    dst = jnp.concatenate(
        [edge_index[1], n + fill % TRASH]).reshape(NW, nchunk, CH)
    cpad = -(-nchunk // STEP) * STEP - nchunk
    src = jnp.pad(src, ((0, 0), (0, cpad), (0, 0)))
    dst = jnp.pad(dst, ((0, 0), (0, cpad), (0, 0)))
    zeros = jnp.zeros((n, hdim), jnp.float32)
    parts = _make_sc_segment_sum(n, hdim, nchunk)(h, src, dst, zeros)

    return _final(parts, h, W_msg, b_msg.reshape(1, hdim), W_out1,
                  b_out1.reshape(1, hdim), W_out2, b_out2.reshape(1, 1),
                  block_rows=1000)
